# Initial kernel scaffold; baseline (speedup 1.0000x reference)
import functools

import jax
import jax.numpy as jnp
from jax import lax
from jax.experimental import pallas as pl
from jax.experimental.pallas import tpu as pltpu
from jax.experimental.pallas import tpu_sc as plsc

VOCAB = 28996
EMBED_DIM = 64

_info = plsc.get_sparse_core_info()
NC, NS, L = _info.num_cores, _info.num_subcores, _info.num_lanes
NW = NC * NS

B_TOTAL = 4096 * 200
CHUNK = 128
N_CHUNKS = B_TOTAL // CHUNK
CPW = N_CHUNKS // NW

_mesh = plsc.VectorSubcoreMesh(core_axis_name="c", subcore_axis_name="s")


@functools.partial(
    pl.kernel,
    out_type=jax.ShapeDtypeStruct((B_TOTAL, EMBED_DIM), jnp.float32),
    mesh=_mesh,
    scratch_types=[
        pltpu.VMEM((CPW, CHUNK), jnp.int32),
        pltpu.VMEM((CHUNK, EMBED_DIM), jnp.float32),
        pltpu.SemaphoreType.DMA,
    ],
)
def _gather_kernel(idx_hbm, table_hbm, out_hbm, idx_v, rows_v, sem):
    wid = lax.axis_index("s") * NC + lax.axis_index("c")
    chunk0 = wid * CPW
    row0 = chunk0 * CHUNK
    pltpu.sync_copy(idx_hbm.at[pl.ds(chunk0, CPW)], idx_v)

    def body(j, carry):
        pltpu.make_async_copy(table_hbm.at[idx_v.at[j]], rows_v, sem).start()
        pltpu.make_async_copy(table_hbm.at[idx_v.at[j]], rows_v, sem).wait()
        pltpu.sync_copy(rows_v, out_hbm.at[pl.ds(row0 + j * CHUNK, CHUNK)])
        return carry

    lax.fori_loop(0, CPW, body, 0)


def kernel(x, table):
    idx = x.reshape(N_CHUNKS, CHUNK)
    out = _gather_kernel(idx, table)
    return out.reshape(4096, 200, EMBED_DIM)


# speedup vs baseline: 3.6812x; 3.6812x over previous
import functools

import jax
import jax.numpy as jnp
from jax import lax
from jax.experimental import pallas as pl
from jax.experimental.pallas import tpu as pltpu
from jax.experimental.pallas import tpu_sc as plsc

VOCAB = 28996
EMBED_DIM = 64

NC, NS, L = 2, 16, 16
NW = NC * NS

B_TOTAL = 4096 * 200
CHUNK = 128
N_CHUNKS = B_TOTAL // CHUNK
CPW = N_CHUNKS // NW

@functools.cache
def _build_gather_kernel():
    mesh = plsc.VectorSubcoreMesh(core_axis_name="c", subcore_axis_name="s")
    return functools.partial(
        pl.kernel,
        out_type=jax.ShapeDtypeStruct((B_TOTAL, EMBED_DIM), jnp.float32),
        mesh=mesh,
        compiler_params=pltpu.CompilerParams(use_tc_tiling_on_sc=False),
        scratch_types=[
            pltpu.VMEM((CPW, CHUNK), jnp.int32),
            pltpu.VMEM((CHUNK, EMBED_DIM), jnp.float32),
            pltpu.SemaphoreType.DMA,
        ],
    )(_gather_body)


def _gather_body(idx_hbm, table_hbm, out_hbm, idx_v, rows_v, sem):
    wid = lax.axis_index("s") * NC + lax.axis_index("c")
    chunk0 = wid * CPW
    row0 = chunk0 * CHUNK
    pltpu.sync_copy(idx_hbm.at[pl.ds(chunk0, CPW)], idx_v)

    def body(j, carry):
        pltpu.make_async_copy(table_hbm.at[idx_v.at[j]], rows_v, sem).start()
        pltpu.make_async_copy(table_hbm.at[idx_v.at[j]], rows_v, sem).wait()
        pltpu.sync_copy(rows_v, out_hbm.at[pl.ds(row0 + j * CHUNK, CHUNK)])
        return carry

    lax.fori_loop(0, CPW, body, 0)


def kernel(x, table):
    idx = x.reshape(N_CHUNKS, CHUNK)
    out = _build_gather_kernel()(idx, table)
    return out.reshape(4096, 200, EMBED_DIM)


# speedup vs baseline: 4.4051x; 1.1967x over previous
import functools

import jax
import jax.numpy as jnp
from jax import lax
from jax.experimental import pallas as pl
from jax.experimental.pallas import tpu as pltpu
from jax.experimental.pallas import tpu_sc as plsc

VOCAB = 28996
EMBED_DIM = 64

NC, NS, L = 2, 16, 16
NW = NC * NS

B_TOTAL = 4096 * 200
CHUNK = 128
N_CHUNKS = B_TOTAL // CHUNK
CPW = N_CHUNKS // NW

NBUF = 4
N_GROUPS = CPW // NBUF


@functools.cache
def _build_gather_kernel():
    mesh = plsc.VectorSubcoreMesh(core_axis_name="c", subcore_axis_name="s")
    return functools.partial(
        pl.kernel,
        out_type=jax.ShapeDtypeStruct((B_TOTAL, EMBED_DIM), jnp.float32),
        mesh=mesh,
        compiler_params=pltpu.CompilerParams(use_tc_tiling_on_sc=False),
        scratch_types=[
            pltpu.VMEM((CPW, CHUNK), jnp.int32),
            pltpu.VMEM((NBUF, CHUNK, EMBED_DIM), jnp.float32),
            [pltpu.SemaphoreType.DMA] * NBUF,
            [pltpu.SemaphoreType.DMA] * NBUF,
        ],
    )(_gather_body)


def _gather_body(idx_hbm, table_hbm, out_hbm, idx_v, rows_v, gsems, osems):
    wid = lax.axis_index("s") * NC + lax.axis_index("c")
    chunk0 = wid * CPW
    row0 = chunk0 * CHUNK

    pltpu.sync_copy(idx_hbm.at[pl.ds(chunk0, CPW)], idx_v)

    def gather(j, b):
        return pltpu.make_async_copy(
            table_hbm.at[idx_v.at[j]], rows_v.at[b], gsems[b]
        )

    def out_copy(j, b):
        return pltpu.make_async_copy(
            rows_v.at[b], out_hbm.at[pl.ds(row0 + j * CHUNK, CHUNK)], osems[b]
        )

    for b in range(NBUF):
        gather(b, b).start()
    for b in range(NBUF):
        gather(b, b).wait()
        out_copy(b, b).start()

    def group(g, carry):
        for b in range(NBUF):
            j = g * NBUF + b
            out_copy(j - NBUF, b).wait()
            gather(j, b).start()
        for b in range(NBUF):
            j = g * NBUF + b
            gather(j, b).wait()
            out_copy(j, b).start()
        return carry

    lax.fori_loop(1, N_GROUPS, group, 0)

    for b in range(NBUF):
        out_copy((N_GROUPS - 1) * NBUF + b, b).wait()


def kernel(x, table):
    idx = x.reshape(N_CHUNKS, CHUNK)
    out = _build_gather_kernel()(idx, table)
    return out.reshape(4096, 200, EMBED_DIM)


# speedup vs baseline: 4.4105x; 1.0012x over previous
import functools

import jax
import jax.numpy as jnp
from jax import lax
from jax.experimental import pallas as pl
from jax.experimental.pallas import tpu as pltpu
from jax.experimental.pallas import tpu_sc as plsc

VOCAB = 28996
EMBED_DIM = 64

NC, NS, L = 2, 16, 16
NW = NC * NS

B_TOTAL = 4096 * 200
CHUNK = 128
N_CHUNKS = B_TOTAL // CHUNK
CPW = N_CHUNKS // NW

NBUF = 8
N_GROUPS = CPW // NBUF


@functools.cache
def _build_gather_kernel():
    mesh = plsc.VectorSubcoreMesh(core_axis_name="c", subcore_axis_name="s")
    return functools.partial(
        pl.kernel,
        out_type=jax.ShapeDtypeStruct((B_TOTAL, EMBED_DIM), jnp.float32),
        mesh=mesh,
        compiler_params=pltpu.CompilerParams(use_tc_tiling_on_sc=False),
        scratch_types=[
            pltpu.VMEM((CPW, CHUNK), jnp.int32),
            pltpu.VMEM((NBUF, CHUNK, EMBED_DIM), jnp.float32),
            [pltpu.SemaphoreType.DMA] * NBUF,
            [pltpu.SemaphoreType.DMA] * NBUF,
        ],
    )(_gather_body)


def _gather_body(idx_hbm, table_hbm, out_hbm, idx_v, rows_v, gsems, osems):
    wid = lax.axis_index("s") * NC + lax.axis_index("c")
    chunk0 = wid * CPW
    row0 = chunk0 * CHUNK

    pltpu.sync_copy(idx_hbm.at[pl.ds(chunk0, CPW)], idx_v)

    def gather(j, b):
        return pltpu.make_async_copy(
            table_hbm.at[idx_v.at[j]], rows_v.at[b], gsems[b]
        )

    def out_copy(j, b):
        return pltpu.make_async_copy(
            rows_v.at[b], out_hbm.at[pl.ds(row0 + j * CHUNK, CHUNK)], osems[b]
        )

    for b in range(NBUF):
        gather(b, b).start()
    for b in range(NBUF):
        gather(b, b).wait()
        out_copy(b, b).start()

    def group(g, carry):
        for b in range(NBUF):
            j = g * NBUF + b
            out_copy(j - NBUF, b).wait()
            gather(j, b).start()
        for b in range(NBUF):
            j = g * NBUF + b
            gather(j, b).wait()
            out_copy(j, b).start()
        return carry

    lax.fori_loop(1, N_GROUPS, group, 0)

    for b in range(NBUF):
        out_copy((N_GROUPS - 1) * NBUF + b, b).wait()


def kernel(x, table):
    idx = x.reshape(N_CHUNKS, CHUNK)
    out = _build_gather_kernel()(idx, table)
    return out.reshape(4096, 200, EMBED_DIM)
